# baseline (speedup 1.0000x reference)
import functools

import jax
import jax.numpy as jnp
from jax import lax
from jax.experimental import pallas as pl
from jax.experimental.pallas import tpu as pltpu
from jax.experimental.pallas import tpu_sc as plsc

NC = 2
NS = 16
NW = NC * NS
CH = 128


def _transpose_kernel(V, D):
    mesh = plsc.VectorSubcoreMesh(core_axis_name="c", subcore_axis_name="s")
    n_full = V // CH
    tail = V - n_full * CH
    base_cnt = n_full // NW
    extra = n_full - base_cnt * NW

    @functools.partial(
        pl.kernel,
        mesh=mesh,
        compiler_params=pltpu.CompilerParams(needs_layout_passes=False),
        out_type=jax.ShapeDtypeStruct((V // 2, 2 * D), jnp.float32),
        scratch_types=[
            pltpu.VMEM((2, D, CH), jnp.float32),
            pltpu.VMEM((2, D, CH), jnp.float32),
            pltpu.SemaphoreType.DMA,
            pltpu.SemaphoreType.DMA,
        ],
    )
    def k(tt_hbm, tail_hbm, out_hbm, ibuf, obuf, isem, osem):
        wid = lax.axis_index("s") * NC + lax.axis_index("c")
        cnt = jnp.where(wid < extra, base_cnt + 1, base_cnt)
        start = wid * base_cnt + jnp.minimum(wid, extra)
        lanes = lax.iota(jnp.int32, 16)

        def start_slab(s, slot):
            for tr in range(D // 8):
                pltpu.async_copy(
                    tt_hbm.at[pl.ds(tr * 8, 8), pl.ds(s * CH, CH)],
                    ibuf.at[slot, pl.ds(tr * 8, 8)],
                    isem,
                )

        def wait_slab(s, slot):
            for tr in range(D // 8):
                pltpu.make_async_copy(
                    tt_hbm.at[pl.ds(tr * 8, 8), pl.ds(s * CH, CH)],
                    ibuf.at[slot, pl.ds(tr * 8, 8)],
                    isem,
                ).wait()

        def do_slab(s, slot):
            wait_slab(s, slot)
            slot16 = jnp.full((16,), slot, jnp.int32)

            def row(vv, _):
                for h in range(2):
                    c16 = jnp.full((16,), 2 * vv + h, jnp.int32)
                    for q in range(D // 16):
                        v = plsc.load_gather(
                            ibuf, [slot16, q * 16 + lanes, c16]
                        )
                        obuf[slot, vv, pl.ds(h * D + q * 16, 16)] = v
                return 0

            lax.fori_loop(0, CH // 2, row, 0, unroll=2)
            pltpu.async_copy(
                obuf.at[slot, pl.ds(0, CH // 2)],
                out_hbm.at[pl.ds(s * (CH // 2), CH // 2)],
                osem,
            )

        def drain_slab(s, slot):
            pltpu.make_async_copy(
                obuf.at[slot, pl.ds(0, CH // 2)],
                out_hbm.at[pl.ds(s * (CH // 2), CH // 2)],
                osem,
            ).wait()

        start_slab(start, 0)

        def body(j, _):
            slot = lax.rem(j, 2)

            @pl.when(j + 1 < cnt)
            def _():
                start_slab(start + j + 1, lax.rem(j + 1, 2))

            @pl.when(j >= 2)
            def _():
                drain_slab(start + j - 2, slot)

            do_slab(start + j, slot)
            return 0

        lax.fori_loop(0, base_cnt + 1, lambda j, c: lax.cond(
            j < cnt, lambda: body(j, c), lambda: 0), 0)

        @pl.when(cnt >= 2)
        def _():
            drain_slab(start + cnt - 2, lax.rem(cnt - 2, 2))
        drain_slab(start + cnt - 1, lax.rem(cnt - 1, 2))

        if tail:
            @pl.when(wid == NW - 1)
            def _():
                pltpu.sync_copy(tail_hbm, obuf.at[0, pl.ds(0, tail // 2)])
                pltpu.sync_copy(
                    obuf.at[0, pl.ds(0, tail // 2)],
                    out_hbm.at[pl.ds(n_full * (CH // 2), tail // 2)],
                )

    return k


def _gather_kernel(B, D, n_chunks):
    mesh = plsc.VectorSubcoreMesh(core_axis_name="c", subcore_axis_name="s")
    D2 = 2 * D
    NBUF = 2

    @functools.partial(
        pl.kernel,
        mesh=mesh,
        compiler_params=pltpu.CompilerParams(needs_layout_passes=False),
        out_type=jax.ShapeDtypeStruct((B, D), jnp.float32),
        scratch_types=[
            pltpu.VMEM((n_chunks, CH), jnp.int32),
            pltpu.VMEM((n_chunks, CH), jnp.int32),
            pltpu.VMEM((NBUF, CH, D2), jnp.float32),
            pltpu.VMEM((2, CH, D), jnp.float32),
            pltpu.SemaphoreType.DMA,
            pltpu.SemaphoreType.DMA,
        ],
    )
    def k(vidx_hbm, px_hbm, tab_hbm, out_hbm,
          idx_v, px_v, buf, outb, gsem, osem):
        wid = lax.axis_index("s") * NC + lax.axis_index("c")
        base = wid * (n_chunks * CH)
        pltpu.sync_copy(vidx_hbm.at[wid], idx_v)
        pltpu.sync_copy(px_hbm.at[wid], px_v)
        lanes = lax.iota(jnp.int32, 16)

        def start_chunk(j, slot):
            pltpu.async_copy(tab_hbm.at[idx_v.at[j]], buf.at[slot], gsem)

        def finish_chunk(j, slot, oslot):
            pltpu.make_async_copy(
                tab_hbm.at[idx_v.at[j]], buf.at[slot], gsem
            ).wait()
            slot16 = jnp.full((16,), slot, jnp.int32)
            j16 = jnp.full((16,), j, jnp.int32)

            def scale_row(r, _):
                r16 = jnp.full((16,), r, jnp.int32)
                p16 = plsc.load_gather(px_v, [j16, r16])
                off = (p16 << 6) + lanes
                for c in range(D // 16):
                    v = plsc.load_gather(buf, [slot16, r16, off + c * 16])
                    outb[oslot, r, pl.ds(c * 16, 16)] = v * 8.0
                return 0

            lax.fori_loop(0, CH, scale_row, 0, unroll=2)
            pltpu.async_copy(
                outb.at[oslot], out_hbm.at[pl.ds(base + j * CH, CH)], osem
            )

        def drain_out(j, oslot):
            pltpu.make_async_copy(
                outb.at[oslot], out_hbm.at[pl.ds(base + j * CH, CH)], osem
            ).wait()

        for j in range(NBUF - 1):
            start_chunk(j, j)

        def body(j, _):
            slot = lax.rem(j, NBUF)
            oslot = lax.rem(j, 2)

            @pl.when(j + NBUF - 1 < n_chunks)
            def _():
                start_chunk(j + NBUF - 1, lax.rem(j + NBUF - 1, NBUF))

            @pl.when(j >= 2)
            def _():
                drain_out(j - 2, oslot)

            finish_chunk(j, slot, oslot)
            return 0

        lax.fori_loop(0, n_chunks, body, 0)
        drain_out(n_chunks - 2, lax.rem(n_chunks - 2, 2))
        drain_out(n_chunks - 1, lax.rem(n_chunks - 1, 2))

    return k


def kernel(x, table):
    S0, S1 = x.shape
    V, D = table.shape
    B = S0 * S1
    n_chunks = B // (NW * CH)
    xi = x.astype(jnp.int32)
    vidx = (xi >> 1).reshape(NW, n_chunks, CH)
    px = (xi & 1).reshape(NW, n_chunks, CH)
    n_full = V // CH
    tail_rm = table[n_full * CH:].reshape((V - n_full * CH) // 2, 2 * D)
    t_rm = _transpose_kernel(V, D)(table.T, tail_rm)
    out = _gather_kernel(B, D, n_chunks)(vidx, px, t_rm)
    return out.reshape(S0, S1, D)


# speedup vs baseline: 1.1219x; 1.1219x over previous
import functools

import jax
import jax.numpy as jnp
from jax import lax
from jax.experimental import pallas as pl
from jax.experimental.pallas import tpu as pltpu
from jax.experimental.pallas import tpu_sc as plsc

NC = 2
NS = 16
NW = NC * NS
CH = 128


def _transpose_kernel(V, D):
    mesh = plsc.VectorSubcoreMesh(core_axis_name="c", subcore_axis_name="s")
    n_full = V // CH
    tail = V - n_full * CH
    base_cnt = n_full // NW
    extra = n_full - base_cnt * NW
    NB = 4

    @functools.partial(
        pl.kernel,
        mesh=mesh,
        compiler_params=pltpu.CompilerParams(needs_layout_passes=False),
        out_type=jax.ShapeDtypeStruct((V // 2, 2 * D), jnp.float32),
        scratch_types=[
            pltpu.VMEM((NB, D, CH), jnp.float32),
            pltpu.VMEM((NB, CH // 2, 2 * D), jnp.float32),
            pltpu.SemaphoreType.DMA,
            pltpu.SemaphoreType.DMA,
        ],
    )
    def k(tt_hbm, tail_hbm, out_hbm, ibuf, obuf, isem, osem):
        wid = lax.axis_index("s") * NC + lax.axis_index("c")
        cnt = jnp.where(wid < extra, base_cnt + 1, base_cnt)
        start = wid * base_cnt + jnp.minimum(wid, extra)
        lanes = lax.iota(jnp.int32, 16)
        i1q = [(q * 16 + lanes) >> 1 for q in range(CH // 16)]
        i2q = [((q * 16 + lanes) & 1) * D for q in range(CH // 16)]

        def start_slab(s, slot):
            for tr in range(D // 8):
                pltpu.async_copy(
                    tt_hbm.at[pl.ds(tr * 8, 8), pl.ds(s * CH, CH)],
                    ibuf.at[slot, pl.ds(tr * 8, 8)],
                    isem,
                )

        def wait_slab(s, slot):
            for tr in range(D // 8):
                pltpu.make_async_copy(
                    tt_hbm.at[pl.ds(tr * 8, 8), pl.ds(s * CH, CH)],
                    ibuf.at[slot, pl.ds(tr * 8, 8)],
                    isem,
                ).wait()

        def do_slab(s, slot):
            wait_slab(s, slot)
            slot16 = jnp.full((16,), slot, jnp.int32)

            def row(c, _):
                c16 = jnp.full((16,), c, jnp.int32)
                for q in range(CH // 16):
                    v = ibuf[slot, c, pl.ds(q * 16, 16)]
                    plsc.store_scatter(
                        obuf, [slot16, i1q[q], i2q[q] + c16], v
                    )
                return 0

            lax.fori_loop(0, D, row, 0, unroll=2)
            pltpu.async_copy(
                obuf.at[slot],
                out_hbm.at[pl.ds(s * (CH // 2), CH // 2)],
                osem,
            )

        def drain_slab(s, slot):
            pltpu.make_async_copy(
                obuf.at[slot],
                out_hbm.at[pl.ds(s * (CH // 2), CH // 2)],
                osem,
            ).wait()

        for u in range(NB - 1):
            @pl.when(u < cnt)
            def _(u=u):
                start_slab(start + u, u)

        def body(j, _):
            for u in range(NB):
                jj = j * NB + u

                @pl.when(jj < cnt)
                def _(jj=jj, u=u):
                    @pl.when(jj + NB - 1 < cnt)
                    def _():
                        start_slab(jj + NB - 1 + start, (u + NB - 1) % NB)

                    @pl.when(jj >= NB)
                    def _():
                        drain_slab(jj - NB + start, u)

                    do_slab(jj + start, u)
            return 0

        lax.fori_loop(0, (base_cnt + 1 + NB - 1) // NB, body, 0)
        for u in range(NB):
            @pl.when(cnt - NB + u >= 0)
            def _(u=u):
                drain_slab(
                    start + cnt - NB + u, lax.rem(cnt - NB + u, NB)
                )

        if tail:
            @pl.when(wid == NW - 1)
            def _():
                pltpu.sync_copy(tail_hbm, obuf.at[0, pl.ds(0, tail // 2)])
                pltpu.sync_copy(
                    obuf.at[0, pl.ds(0, tail // 2)],
                    out_hbm.at[pl.ds(n_full * (CH // 2), tail // 2)],
                )

    return k


def _gather_kernel(B, V, D, n_chunks):
    mesh = plsc.VectorSubcoreMesh(core_axis_name="c", subcore_axis_name="s")
    NBUF = 4
    NSPLIT = 4
    SUB = CH // NSPLIT
    D2 = 2 * D

    @functools.partial(
        pl.kernel,
        mesh=mesh,
        compiler_params=pltpu.CompilerParams(needs_layout_passes=False),
        out_type=jax.ShapeDtypeStruct((B, D), jnp.float32),
        scratch_types=[
            pltpu.VMEM((n_chunks, CH), jnp.int32),
            pltpu.VMEM((NBUF, CH), jnp.int32),
            pltpu.VMEM((NBUF, CH, D2), jnp.float32),
            pltpu.VMEM((2, CH, D), jnp.float32),
            pltpu.SemaphoreType.DMA,
            pltpu.SemaphoreType.DMA,
            pltpu.SemaphoreType.DMA,
        ],
    )
    def k(idx_hbm, px_hbm, tab_hbm, out_hbm,
          idx_v, px_v, buf, outb, gsem, psem, osem):
        wid = lax.axis_index("s") * NC + lax.axis_index("c")
        base = wid * (n_chunks * CH)
        pltpu.sync_copy(idx_hbm.at[wid], idx_v)
        lanes = lax.iota(jnp.int32, 16)

        def start_chunk(j, slot):
            pltpu.async_copy(px_hbm.at[wid, j], px_v.at[slot], psem)
            for t in range(NSPLIT):
                pltpu.async_copy(
                    tab_hbm.at[idx_v.at[j, pl.ds(t * SUB, SUB)]],
                    buf.at[slot, pl.ds(t * SUB, SUB)],
                    gsem,
                )

        def finish_chunk(j, slot, oslot):
            pltpu.make_async_copy(px_hbm.at[wid, j], px_v.at[slot], psem).wait()
            for t in range(NSPLIT):
                pltpu.make_async_copy(
                    tab_hbm.at[idx_v.at[j, pl.ds(t * SUB, SUB)]],
                    buf.at[slot, pl.ds(t * SUB, SUB)],
                    gsem,
                ).wait()
            slot16 = jnp.full((16,), slot, jnp.int32)

            def scale_row(r, _):
                r16 = jnp.full((16,), r, jnp.int32)
                p16 = plsc.load_gather(px_v, [slot16, r16])
                off = (p16 << 6) + lanes
                for c in range(D // 16):
                    v = plsc.load_gather(buf, [slot16, r16, off + c * 16])
                    outb[oslot, r, pl.ds(c * 16, 16)] = v * 8.0
                return 0

            lax.fori_loop(0, CH, scale_row, 0, unroll=2)
            pltpu.async_copy(
                outb.at[oslot], out_hbm.at[pl.ds(base + j * CH, CH)], osem
            )

        def drain_out(j, oslot):
            pltpu.make_async_copy(
                outb.at[oslot], out_hbm.at[pl.ds(base + j * CH, CH)], osem
            ).wait()

        for u in range(NBUF - 1):
            start_chunk(u, u)

        def body(j, _):
            for u in range(NBUF):
                jj = j * NBUF + u
                oslot = lax.rem(jj, 2)

                @pl.when(jj + NBUF - 1 < n_chunks)
                def _(jj=jj, u=u):
                    start_chunk(jj + NBUF - 1, (u + NBUF - 1) % NBUF)

                @pl.when(jj >= 2)
                def _(jj=jj, oslot=oslot):
                    drain_out(jj - 2, oslot)

                finish_chunk(jj, u, oslot)
            return 0

        lax.fori_loop(0, n_chunks // NBUF, body, 0)
        drain_out(n_chunks - 2, lax.rem(n_chunks - 2, 2))
        drain_out(n_chunks - 1, lax.rem(n_chunks - 1, 2))

    return k


def kernel(x, table):
    S0, S1 = x.shape
    V, D = table.shape
    B = S0 * S1
    n_chunks = B // (NW * CH)
    xi = x.astype(jnp.int32)
    vidx = (xi >> 1).reshape(NW, n_chunks, CH)
    px = (xi & 1).reshape(NW, n_chunks, CH)
    n_full = V // CH
    tail_rm = table[n_full * CH:].reshape((V - n_full * CH) // 2, 2 * D)
    t_rm = _transpose_kernel(V, D)(table.T, tail_rm)
    out = _gather_kernel(B, V, D, n_chunks)(vidx, px, t_rm)
    return out.reshape(S0, S1, D)
